# baseline (speedup 1.0000x reference)
import jax
import jax.numpy as jnp
from jax import lax
from jax.experimental import pallas as pl
from jax.experimental.pallas import tpu as pltpu
from jax.experimental.pallas import tpu_sc as plsc

N = 100000
D = 128
K = 64
BLK = 1024
GRID = (N + BLK - 1) // BLK
CAND = GRID * K
NSUB = 32
CH_PER = D // NSUB

NEG = float(-jnp.finfo(jnp.float32).max)
MXU_MIN_D = 8


def _ce_lane(x, d, take_min, low):
    a = pltpu.roll(x, D - d, 1)
    b = pltpu.roll(x, d, 1)
    xp = jnp.where(low, a, b)
    return jnp.where(take_min, jnp.minimum(x, xp), jnp.maximum(x, xp))


def _xor_perms():
    r = jax.lax.broadcasted_iota(jnp.int32, (D, D), 0)
    c = jax.lax.broadcasted_iota(jnp.int32, (D, D), 1)
    perms = {}
    d = MXU_MIN_D
    while d < D:
        perms[d] = jnp.where(c == (r ^ d), 1.0, 0.0).astype(jnp.float32)
        d *= 2
    return perms


def _ce_lane_mxu(x, pd, take_min):
    xp = jax.lax.dot_general(
        x, pd, (((1,), (0,)), ((), ())),
        precision=jax.lax.Precision.HIGHEST,
        preferred_element_type=jnp.float32)
    return jnp.where(take_min, jnp.minimum(x, xp), jnp.maximum(x, xp))


def _ce_sub(x, d, take_max, low):
    a = pltpu.roll(x, x.shape[0] - d, 0)
    b = pltpu.roll(x, d, 0)
    xp = jnp.where(low, a, b)
    return jnp.where(take_max, jnp.maximum(x, xp), jnp.minimum(x, xp))


def _stages(n):
    out = []
    kk = 2
    while kk <= n:
        d = kk // 2
        while d >= 1:
            out.append((kk, d))
            d //= 2
        kk *= 2
    return out


def _lane_masks():
    lane = jax.lax.broadcasted_iota(jnp.int32, (1, D), 1)
    masks = {}
    for kk, d in _stages(D):
        desc = (lane & kk) != 0
        low = (lane & d) == 0
        masks[(kk, d)] = (jnp.logical_xor(low, desc), low)
    return masks


def _alt_sort_masks(rows):
    r = jax.lax.broadcasted_iota(jnp.int32, (rows, 1), 0)
    odd = (r & K) != 0
    rl = r & (K - 1)
    masks = {}
    for kk, d in _stages(K):
        blk = (rl & kk) != 0
        low = (rl & d) == 0
        m = jnp.logical_xor(jnp.logical_xor(low, blk), odd)
        masks[(kk, d)] = (m, low)
    return masks


def _alt_clean_masks(rows):
    r = jax.lax.broadcasted_iota(jnp.int32, (rows, 1), 0)
    odd = (r & K) != 0
    masks = {}
    d = K // 2
    while d >= 1:
        low = (r & d) == 0
        masks[d] = (jnp.logical_xor(low, odd), low)
        d //= 2
    return masks


def _sort_rows_asc(x, lm, perms):
    for kk, d in _stages(D):
        take_min, low = lm[(kk, d)]
        if d >= MXU_MIN_D:
            x = _ce_lane_mxu(x, perms[d], take_min)
        else:
            x = _ce_lane(x, d, take_min, low)
    return x


def _clean_runs(c, cm):
    d = K // 2
    while d >= 1:
        take_max, low = cm[d]
        c = _ce_sub(c, d, take_max, low)
        d //= 2
    return c


G = BLK // K
GROUP = G * K


def _group_top64(xt, alt_sort, alt_cleans, low64, final_desc):
    for kk, d in _stages(K):
        take_max, low = alt_sort[(kk, d)]
        xt = _ce_sub(xt, d, take_max, low)
    rows = GROUP
    while rows > K:
        half = rows // 2
        ntile = rows // K
        a = jnp.concatenate(
            [xt[2 * j * K:(2 * j + 1) * K, :] for j in range(ntile // 2)], 0)
        b = jnp.concatenate(
            [xt[(2 * j + 1) * K:(2 * j + 2) * K, :] for j in range(ntile // 2)], 0)
        c = jnp.maximum(a, b)
        if half > K:
            xt = _clean_runs(c, alt_cleans[half])
        else:
            cm = {d: ((low64[d] if final_desc else jnp.logical_not(low64[d])),
                      low64[d]) for d in low64}
            xt = _clean_runs(c, cm)
        rows = half
    return xt


def _tc_body(x_ref, o_ref):
    i = pl.program_id(0)
    lm = _lane_masks()
    perms = _xor_perms()
    alt_sort = _alt_sort_masks(GROUP)
    alt_cleans = {r: _alt_clean_masks(r)
                  for r in (GROUP // 2, GROUP // 4, GROUP // 8, K)
                  if r > K}
    r_grp = jax.lax.broadcasted_iota(jnp.int32, (GROUP, 1), 0)
    r64 = jax.lax.broadcasted_iota(jnp.int32, (K, 1), 0)
    low64 = {}
    d = K // 2
    while d >= 1:
        low64[d] = (r64 & d) == 0
        d //= 2

    xt = x_ref[...]
    rowg = r_grp + i * BLK
    xt = jnp.where(rowg < N, xt, NEG)
    xt = _sort_rows_asc(xt, lm, perms)
    o_ref[...] = _group_top64(xt, alt_sort, alt_cleans, low64,
                              final_desc=True)


def _run_tc(feat, interpret=False):
    return pl.pallas_call(
        _tc_body,
        grid=(GRID,),
        in_specs=[pl.BlockSpec((BLK, D), lambda i: (i, 0))],
        out_specs=pl.BlockSpec((K, D), lambda i: (i, 0)),
        out_shape=jax.ShapeDtypeStruct((CAND, D), jnp.float32),
        compiler_params=pltpu.CompilerParams(
            dimension_semantics=("parallel",)),
        interpret=interpret,
    )(feat)


def _ce16_desc(z, d):
    i16 = lax.iota(jnp.int32, 16)
    p = jnp.take_along_axis(z, i16 ^ d, axis=0, mode="promise_in_bounds")
    low = (i16 & d) == 0
    return jnp.where(low, jnp.maximum(z, p), jnp.minimum(z, p))


def _sc_merge_desc(best, run):
    rev = [lax.rev(run[3 - t], (0,)) for t in range(4)]
    c = [jnp.maximum(best[t], rev[t]) for t in range(4)]
    y0, y2 = jnp.maximum(c[0], c[2]), jnp.minimum(c[0], c[2])
    y1, y3 = jnp.maximum(c[1], c[3]), jnp.minimum(c[1], c[3])
    z0, z1 = jnp.maximum(y0, y1), jnp.minimum(y0, y1)
    z2, z3 = jnp.maximum(y2, y3), jnp.minimum(y2, y3)
    out = []
    for z in (z0, z1, z2, z3):
        for d in (8, 4, 2, 1):
            z = _ce16_desc(z, d)
        out.append(z)
    return tuple(out)


def _sc_fold(cand_hbm, out_hbm, colbuf, outbuf):
    wid = lax.axis_index("s") * 2 + lax.axis_index("c")
    bc = wid * CH_PER
    pltpu.sync_copy(cand_hbm.at[pl.ds(bc, CH_PER)], colbuf)
    for j in range(CH_PER):
        best = tuple(colbuf[j, 16 * t:16 * (t + 1)] for t in range(4))

        def body(r, b, j=j):
            run = tuple(colbuf[j, pl.ds(r * K + 16 * t, 16)]
                        for t in range(4))
            return _sc_merge_desc(b, run)

        best = lax.fori_loop(1, GRID, body, best)
        for t in range(4):
            outbuf[j, 16 * t:16 * (t + 1)] = best[t]
    pltpu.sync_copy(outbuf, out_hbm.at[pl.ds(bc, CH_PER)])


def _run_sc(cand_t):
    mesh = plsc.VectorSubcoreMesh(core_axis_name="c", subcore_axis_name="s")
    f = pl.kernel(
        _sc_fold,
        out_type=jax.ShapeDtypeStruct((D, K), jnp.float32),
        mesh=mesh,
        scratch_types=[
            pltpu.VMEM((CH_PER, CAND), jnp.float32),
            pltpu.VMEM((CH_PER, K), jnp.float32),
        ],
    )
    return f(cand_t)


@jax.jit
def kernel(feat):
    cand = _run_tc(feat)
    scout = _run_sc(cand.T)
    return scout.T.reshape(K * D)


# speedup vs baseline: 1.3867x; 1.3867x over previous
import jax
import jax.numpy as jnp
from jax import lax
from jax.experimental import pallas as pl
from jax.experimental.pallas import tpu as pltpu
from jax.experimental.pallas import tpu_sc as plsc

N = 100000
D = 128
K = 64
BLK = 1024
GRID = (N + BLK - 1) // BLK
CAND = GRID * K
NSUB = 32
CH_PER = D // NSUB

NEG = float(-jnp.finfo(jnp.float32).max)
MXU_MIN_D = D + 1


def _ce_lane(x, d, take_min, low):
    a = pltpu.roll(x, D - d, 1)
    b = pltpu.roll(x, d, 1)
    xp = jnp.where(low, a, b)
    return jnp.where(take_min, jnp.minimum(x, xp), jnp.maximum(x, xp))


def _xor_perms():
    r = jax.lax.broadcasted_iota(jnp.int32, (D, D), 0)
    c = jax.lax.broadcasted_iota(jnp.int32, (D, D), 1)
    perms = {}
    d = MXU_MIN_D
    while d < D:
        perms[d] = jnp.where(c == (r ^ d), 1.0, 0.0).astype(jnp.float32)
        d *= 2
    return perms


def _ce_lane_mxu(x, pd, take_min):
    xp = jax.lax.dot_general(
        x, pd, (((1,), (0,)), ((), ())),
        precision=jax.lax.Precision.HIGHEST,
        preferred_element_type=jnp.float32)
    return jnp.where(take_min, jnp.minimum(x, xp), jnp.maximum(x, xp))


def _ce_sub(x, d, take_max, low):
    a = pltpu.roll(x, x.shape[0] - d, 0)
    b = pltpu.roll(x, d, 0)
    xp = jnp.where(low, a, b)
    return jnp.where(take_max, jnp.maximum(x, xp), jnp.minimum(x, xp))


def _stages(n):
    out = []
    kk = 2
    while kk <= n:
        d = kk // 2
        while d >= 1:
            out.append((kk, d))
            d //= 2
        kk *= 2
    return out


def _lane_masks():
    lane = jax.lax.broadcasted_iota(jnp.int32, (1, D), 1)
    masks = {}
    for kk, d in _stages(D):
        desc = (lane & kk) != 0
        low = (lane & d) == 0
        masks[(kk, d)] = (jnp.logical_xor(low, desc), low)
    return masks


def _alt_sort_masks(rows):
    r = jax.lax.broadcasted_iota(jnp.int32, (rows, 1), 0)
    odd = (r & K) != 0
    rl = r & (K - 1)
    masks = {}
    for kk, d in _stages(K):
        blk = (rl & kk) != 0
        low = (rl & d) == 0
        m = jnp.logical_xor(jnp.logical_xor(low, blk), odd)
        masks[(kk, d)] = (m, low)
    return masks


def _alt_clean_masks(rows):
    r = jax.lax.broadcasted_iota(jnp.int32, (rows, 1), 0)
    odd = (r & K) != 0
    masks = {}
    d = K // 2
    while d >= 1:
        low = (r & d) == 0
        masks[d] = (jnp.logical_xor(low, odd), low)
        d //= 2
    return masks


def _sort_rows_asc(x, lm, perms):
    for kk, d in _stages(D):
        take_min, low = lm[(kk, d)]
        if d >= MXU_MIN_D:
            x = _ce_lane_mxu(x, perms[d], take_min)
        else:
            x = _ce_lane(x, d, take_min, low)
    return x


def _clean_runs(c, cm):
    d = K // 2
    while d >= 1:
        take_max, low = cm[d]
        c = _ce_sub(c, d, take_max, low)
        d //= 2
    return c


G = BLK // K
GROUP = G * K


def _group_top64(xt, alt_sort, alt_cleans, low64, final_desc):
    for kk, d in _stages(K):
        take_max, low = alt_sort[(kk, d)]
        xt = _ce_sub(xt, d, take_max, low)
    rows = GROUP
    while rows > K:
        half = rows // 2
        ntile = rows // K
        a = jnp.concatenate(
            [xt[2 * j * K:(2 * j + 1) * K, :] for j in range(ntile // 2)], 0)
        b = jnp.concatenate(
            [xt[(2 * j + 1) * K:(2 * j + 2) * K, :] for j in range(ntile // 2)], 0)
        c = jnp.maximum(a, b)
        if half > K:
            xt = _clean_runs(c, alt_cleans[half])
        else:
            cm = {d: ((low64[d] if final_desc else jnp.logical_not(low64[d])),
                      low64[d]) for d in low64}
            xt = _clean_runs(c, cm)
        rows = half
    return xt


def _tc_body(x_ref, o_ref):
    i = pl.program_id(0)
    lm = _lane_masks()
    perms = _xor_perms()
    alt_sort = _alt_sort_masks(GROUP)
    alt_cleans = {}
    r = GROUP // 2
    while r > K:
        alt_cleans[r] = _alt_clean_masks(r)
        r //= 2
    r_grp = jax.lax.broadcasted_iota(jnp.int32, (GROUP, 1), 0)
    r64 = jax.lax.broadcasted_iota(jnp.int32, (K, 1), 0)
    low64 = {}
    d = K // 2
    while d >= 1:
        low64[d] = (r64 & d) == 0
        d //= 2

    xt = x_ref[...]
    rowg = r_grp + i * BLK
    xt = jnp.where(rowg < N, xt, NEG)
    xt = _sort_rows_asc(xt, lm, perms)
    o_ref[...] = _group_top64(xt, alt_sort, alt_cleans, low64,
                              final_desc=True)


def _run_tc(feat, interpret=False):
    return pl.pallas_call(
        _tc_body,
        grid=(GRID,),
        in_specs=[pl.BlockSpec((BLK, D), lambda i: (i, 0))],
        out_specs=pl.BlockSpec((K, D), lambda i: (i, 0)),
        out_shape=jax.ShapeDtypeStruct((CAND, D), jnp.float32),
        compiler_params=pltpu.CompilerParams(
            dimension_semantics=("parallel",)),
        interpret=interpret,
    )(feat)


def _ce16_desc(z, d):
    i16 = lax.iota(jnp.int32, 16)
    p = jnp.take_along_axis(z, i16 ^ d, axis=0, mode="promise_in_bounds")
    low = (i16 & d) == 0
    return jnp.where(low, jnp.maximum(z, p), jnp.minimum(z, p))


def _sc_merge_desc(best, run):
    rev = [lax.rev(run[3 - t], (0,)) for t in range(4)]
    c = [jnp.maximum(best[t], rev[t]) for t in range(4)]
    y0, y2 = jnp.maximum(c[0], c[2]), jnp.minimum(c[0], c[2])
    y1, y3 = jnp.maximum(c[1], c[3]), jnp.minimum(c[1], c[3])
    z0, z1 = jnp.maximum(y0, y1), jnp.minimum(y0, y1)
    z2, z3 = jnp.maximum(y2, y3), jnp.minimum(y2, y3)
    out = []
    for z in (z0, z1, z2, z3):
        for d in (8, 4, 2, 1):
            z = _ce16_desc(z, d)
        out.append(z)
    return tuple(out)


def _sc_fold(cand_hbm, out_hbm, colbuf, outbuf):
    wid = lax.axis_index("s") * 2 + lax.axis_index("c")
    bc = wid * CH_PER
    pltpu.sync_copy(cand_hbm.at[pl.ds(bc, CH_PER)], colbuf)
    for j in range(CH_PER):
        best = tuple(colbuf[j, 16 * t:16 * (t + 1)] for t in range(4))

        def body(r, b, j=j):
            run = tuple(colbuf[j, pl.ds(r * K + 16 * t, 16)]
                        for t in range(4))
            return _sc_merge_desc(b, run)

        best = lax.fori_loop(1, GRID, body, best)
        for t in range(4):
            outbuf[j, 16 * t:16 * (t + 1)] = best[t]
    pltpu.sync_copy(outbuf, out_hbm.at[pl.ds(bc, CH_PER)])


def _run_sc(cand_t):
    mesh = plsc.VectorSubcoreMesh(core_axis_name="c", subcore_axis_name="s")
    f = pl.kernel(
        _sc_fold,
        out_type=jax.ShapeDtypeStruct((D, K), jnp.float32),
        mesh=mesh,
        scratch_types=[
            pltpu.VMEM((CH_PER, CAND), jnp.float32),
            pltpu.VMEM((CH_PER, K), jnp.float32),
        ],
    )
    return f(cand_t)


@jax.jit
def kernel(feat):
    cand = _run_tc(feat)
    scout = _run_sc(cand.T)
    return scout.T.reshape(K * D)


# speedup vs baseline: 1.5260x; 1.1004x over previous
import jax
import jax.numpy as jnp
from jax import lax
from jax.experimental import pallas as pl
from jax.experimental.pallas import tpu as pltpu
from jax.experimental.pallas import tpu_sc as plsc

N = 100000
D = 128
K = 64
BLK = 1024
GRID = (N + BLK - 1) // BLK
CAND = GRID * K
NSUB = 32
CH_PER = D // NSUB

NEG = float(-jnp.finfo(jnp.float32).max)


def _ce_lane(x, d, take_min, low):
    a = pltpu.roll(x, D - d, 1)
    b = pltpu.roll(x, d, 1)
    xp = jnp.where(low, a, b)
    return jnp.where(take_min, jnp.minimum(x, xp), jnp.maximum(x, xp))


def _ce_sub(x, d, take_max, low):
    a = pltpu.roll(x, x.shape[0] - d, 0)
    b = pltpu.roll(x, d, 0)
    xp = jnp.where(low, a, b)
    return jnp.where(take_max, jnp.maximum(x, xp), jnp.minimum(x, xp))


def _stages(n):
    out = []
    kk = 2
    while kk <= n:
        d = kk // 2
        while d >= 1:
            out.append((kk, d))
            d //= 2
        kk *= 2
    return out


def _sort_rows_asc(x, lane):
    for kk, d in _stages(D):
        desc = (lane & kk) != 0
        low = (lane & d) == 0
        take_min = jnp.logical_xor(low, desc)
        x = _ce_lane(x, d, take_min, low)
    return x


def _sort64(x, row, desc):
    for kk, d in _stages(K):
        blk = (row & kk) != 0
        low = (row & d) == 0
        m = jnp.logical_xor(low, blk)
        take_max = m if desc else jnp.logical_not(m)
        x = _ce_sub(x, d, take_max, low)
    return x


def _clean64(c, row, desc):
    d = K // 2
    while d >= 1:
        low = (row & d) == 0
        take_max = low if desc else jnp.logical_not(low)
        c = _ce_sub(c, d, take_max, low)
        d //= 2
    return c


def _merge64(a_desc, b_asc, row, desc):
    return _clean64(jnp.maximum(a_desc, b_asc), row, desc)


def _block_top64(tiles, row, desc):
    if len(tiles) == 1:
        return _sort64(tiles[0], row, desc)
    h = len(tiles) // 2
    a = _block_top64(tiles[:h], row, True)
    b = _block_top64(tiles[h:], row, False)
    return _merge64(a, b, row, desc)


def _tc_body(x_ref, o_ref):
    i = pl.program_id(0)
    x = x_ref[...]
    rowg = jax.lax.broadcasted_iota(jnp.int32, (BLK, 1), 0) + i * BLK
    x = jnp.where(rowg < N, x, NEG)
    lane = jax.lax.broadcasted_iota(jnp.int32, (1, D), 1)
    x = _sort_rows_asc(x, lane)

    row = jax.lax.broadcasted_iota(jnp.int32, (K, 1), 0)
    tiles = [x[t * K:(t + 1) * K, :] for t in range(BLK // K)]
    o_ref[...] = _block_top64(tiles, row, desc=True)


def _run_tc(feat, interpret=False):
    return pl.pallas_call(
        _tc_body,
        grid=(GRID,),
        in_specs=[pl.BlockSpec((BLK, D), lambda i: (i, 0))],
        out_specs=pl.BlockSpec((K, D), lambda i: (i, 0)),
        out_shape=jax.ShapeDtypeStruct((CAND, D), jnp.float32),
        compiler_params=pltpu.CompilerParams(
            dimension_semantics=("parallel",)),
        interpret=interpret,
    )(feat)


def _ce16_desc(z, d):
    i16 = lax.iota(jnp.int32, 16)
    p = jnp.take_along_axis(z, i16 ^ d, axis=0, mode="promise_in_bounds")
    low = (i16 & d) == 0
    return jnp.where(low, jnp.maximum(z, p), jnp.minimum(z, p))


def _sc_merge_desc(best, run):
    rev = [lax.rev(run[3 - t], (0,)) for t in range(4)]
    c = [jnp.maximum(best[t], rev[t]) for t in range(4)]
    y0, y2 = jnp.maximum(c[0], c[2]), jnp.minimum(c[0], c[2])
    y1, y3 = jnp.maximum(c[1], c[3]), jnp.minimum(c[1], c[3])
    z0, z1 = jnp.maximum(y0, y1), jnp.minimum(y0, y1)
    z2, z3 = jnp.maximum(y2, y3), jnp.minimum(y2, y3)
    out = []
    for z in (z0, z1, z2, z3):
        for d in (8, 4, 2, 1):
            z = _ce16_desc(z, d)
        out.append(z)
    return tuple(out)


def _sc_fold(cand_hbm, out_hbm, colbuf, outbuf):
    wid = lax.axis_index("s") * 2 + lax.axis_index("c")
    bc = wid * CH_PER
    pltpu.sync_copy(cand_hbm.at[pl.ds(bc, CH_PER)], colbuf)
    for j in range(CH_PER):
        best = tuple(colbuf[j, 16 * t:16 * (t + 1)] for t in range(4))

        def body(r, b, j=j):
            run = tuple(colbuf[j, pl.ds(r * K + 16 * t, 16)]
                        for t in range(4))
            return _sc_merge_desc(b, run)

        best = lax.fori_loop(1, GRID, body, best)
        for t in range(4):
            outbuf[j, 16 * t:16 * (t + 1)] = best[t]
    pltpu.sync_copy(outbuf, out_hbm.at[pl.ds(bc, CH_PER)])


def _run_sc(cand_t):
    mesh = plsc.VectorSubcoreMesh(core_axis_name="c", subcore_axis_name="s")
    f = pl.kernel(
        _sc_fold,
        out_type=jax.ShapeDtypeStruct((D, K), jnp.float32),
        mesh=mesh,
        scratch_types=[
            pltpu.VMEM((CH_PER, CAND), jnp.float32),
            pltpu.VMEM((CH_PER, K), jnp.float32),
        ],
    )
    return f(cand_t)


@jax.jit
def kernel(feat):
    cand = _run_tc(feat)
    scout = _run_sc(cand.T)
    return scout.T.reshape(K * D)


# speedup vs baseline: 1.5363x; 1.0068x over previous
import jax
import jax.numpy as jnp
from jax import lax
from jax.experimental import pallas as pl
from jax.experimental.pallas import tpu as pltpu
from jax.experimental.pallas import tpu_sc as plsc

N = 100000
D = 128
K = 64
BLK = 2048
GRID = (N + BLK - 1) // BLK
CAND = GRID * K
NSUB = 32
CH_PER = D // NSUB

NEG = float(-jnp.finfo(jnp.float32).max)


def _ce_lane(x, d, take_min, low):
    a = pltpu.roll(x, D - d, 1)
    b = pltpu.roll(x, d, 1)
    xp = jnp.where(low, a, b)
    return jnp.where(take_min, jnp.minimum(x, xp), jnp.maximum(x, xp))


def _ce_sub(x, d, take_max, low):
    a = pltpu.roll(x, x.shape[0] - d, 0)
    b = pltpu.roll(x, d, 0)
    xp = jnp.where(low, a, b)
    return jnp.where(take_max, jnp.maximum(x, xp), jnp.minimum(x, xp))


def _stages(n):
    out = []
    kk = 2
    while kk <= n:
        d = kk // 2
        while d >= 1:
            out.append((kk, d))
            d //= 2
        kk *= 2
    return out


def _sort_rows_asc(x, lane):
    for kk, d in _stages(D):
        desc = (lane & kk) != 0
        low = (lane & d) == 0
        take_min = jnp.logical_xor(low, desc)
        x = _ce_lane(x, d, take_min, low)
    return x


def _sort64(x, row, desc):
    for kk, d in _stages(K):
        blk = (row & kk) != 0
        low = (row & d) == 0
        m = jnp.logical_xor(low, blk)
        take_max = m if desc else jnp.logical_not(m)
        x = _ce_sub(x, d, take_max, low)
    return x


def _clean64(c, row, desc):
    d = K // 2
    while d >= 1:
        low = (row & d) == 0
        take_max = low if desc else jnp.logical_not(low)
        c = _ce_sub(c, d, take_max, low)
        d //= 2
    return c


def _merge64(a_desc, b_asc, row, desc):
    return _clean64(jnp.maximum(a_desc, b_asc), row, desc)


def _block_top64(tiles, row, desc):
    if len(tiles) == 1:
        return _sort64(tiles[0], row, desc)
    h = len(tiles) // 2
    a = _block_top64(tiles[:h], row, True)
    b = _block_top64(tiles[h:], row, False)
    return _merge64(a, b, row, desc)


def _tc_body(x_ref, o_ref):
    i = pl.program_id(0)
    x = x_ref[...]
    rowg = jax.lax.broadcasted_iota(jnp.int32, (BLK, 1), 0) + i * BLK
    x = jnp.where(rowg < N, x, NEG)
    lane = jax.lax.broadcasted_iota(jnp.int32, (1, D), 1)
    x = _sort_rows_asc(x, lane)

    row = jax.lax.broadcasted_iota(jnp.int32, (K, 1), 0)
    tiles = [x[t * K:(t + 1) * K, :] for t in range(BLK // K)]
    o_ref[...] = _block_top64(tiles, row, desc=True)


def _run_tc(feat, interpret=False):
    return pl.pallas_call(
        _tc_body,
        grid=(GRID,),
        in_specs=[pl.BlockSpec((BLK, D), lambda i: (i, 0))],
        out_specs=pl.BlockSpec((K, D), lambda i: (i, 0)),
        out_shape=jax.ShapeDtypeStruct((CAND, D), jnp.float32),
        compiler_params=pltpu.CompilerParams(
            dimension_semantics=("parallel",)),
        interpret=interpret,
    )(feat)


def _ce16_desc(z, d):
    i16 = lax.iota(jnp.int32, 16)
    p = jnp.take_along_axis(z, i16 ^ d, axis=0, mode="promise_in_bounds")
    low = (i16 & d) == 0
    return jnp.where(low, jnp.maximum(z, p), jnp.minimum(z, p))


def _sc_merge_desc(best, run):
    rev = [lax.rev(run[3 - t], (0,)) for t in range(4)]
    c = [jnp.maximum(best[t], rev[t]) for t in range(4)]
    y0, y2 = jnp.maximum(c[0], c[2]), jnp.minimum(c[0], c[2])
    y1, y3 = jnp.maximum(c[1], c[3]), jnp.minimum(c[1], c[3])
    z0, z1 = jnp.maximum(y0, y1), jnp.minimum(y0, y1)
    z2, z3 = jnp.maximum(y2, y3), jnp.minimum(y2, y3)
    out = []
    for z in (z0, z1, z2, z3):
        for d in (8, 4, 2, 1):
            z = _ce16_desc(z, d)
        out.append(z)
    return tuple(out)


def _sc_fold(cand_hbm, out_hbm, colbuf, outbuf):
    wid = lax.axis_index("s") * 2 + lax.axis_index("c")
    bc = wid * CH_PER
    pltpu.sync_copy(cand_hbm.at[pl.ds(bc, CH_PER)], colbuf)
    for j in range(CH_PER):
        best = tuple(colbuf[j, 16 * t:16 * (t + 1)] for t in range(4))

        def body(r, b, j=j):
            run = tuple(colbuf[j, pl.ds(r * K + 16 * t, 16)]
                        for t in range(4))
            return _sc_merge_desc(b, run)

        best = lax.fori_loop(1, GRID, body, best)
        for t in range(4):
            outbuf[j, 16 * t:16 * (t + 1)] = best[t]
    pltpu.sync_copy(outbuf, out_hbm.at[pl.ds(bc, CH_PER)])


def _run_sc(cand_t):
    mesh = plsc.VectorSubcoreMesh(core_axis_name="c", subcore_axis_name="s")
    f = pl.kernel(
        _sc_fold,
        out_type=jax.ShapeDtypeStruct((D, K), jnp.float32),
        mesh=mesh,
        scratch_types=[
            pltpu.VMEM((CH_PER, CAND), jnp.float32),
            pltpu.VMEM((CH_PER, K), jnp.float32),
        ],
    )
    return f(cand_t)


@jax.jit
def kernel(feat):
    cand = _run_tc(feat)
    scout = _run_sc(cand.T)
    return scout.T.reshape(K * D)


# speedup vs baseline: 1.5726x; 1.0236x over previous
import jax
import jax.numpy as jnp
from jax import lax
from jax.experimental import pallas as pl
from jax.experimental.pallas import tpu as pltpu
from jax.experimental.pallas import tpu_sc as plsc

N = 100000
D = 128
K = 64
BLK = 2048
GRID = (N + BLK - 1) // BLK
CAND = GRID * K
NSUB = 32
CH_PER = D // NSUB

NEG = float(-jnp.finfo(jnp.float32).max)


def _ce_lane(x, d, take_min, low):
    a = pltpu.roll(x, D - d, 1)
    b = pltpu.roll(x, d, 1)
    xp = jnp.where(low, a, b)
    return jnp.where(take_min, jnp.minimum(x, xp), jnp.maximum(x, xp))


def _ce_sub(x, d, take_max, low):
    a = pltpu.roll(x, x.shape[0] - d, 0)
    b = pltpu.roll(x, d, 0)
    xp = jnp.where(low, a, b)
    return jnp.where(take_max, jnp.maximum(x, xp), jnp.minimum(x, xp))


def _ce_lane_clean(x, d, low):
    a = pltpu.roll(x, D - d, 1)
    b = pltpu.roll(x, d, 1)
    return jnp.where(low, jnp.minimum(x, a), jnp.maximum(x, b))


def _ce_sub_clean(x, d, low, desc):
    a = pltpu.roll(x, x.shape[0] - d, 0)
    b = pltpu.roll(x, d, 0)
    if desc:
        return jnp.where(low, jnp.maximum(x, a), jnp.minimum(x, b))
    return jnp.where(low, jnp.minimum(x, a), jnp.maximum(x, b))


def _stages(n):
    out = []
    kk = 2
    while kk <= n:
        d = kk // 2
        while d >= 1:
            out.append((kk, d))
            d //= 2
        kk *= 2
    return out


def _sort_rows_asc(x, lane):
    for kk, d in _stages(D):
        low = (lane & d) == 0
        if kk == D:
            x = _ce_lane_clean(x, d, low)
        else:
            desc = (lane & kk) != 0
            take_min = jnp.logical_xor(low, desc)
            x = _ce_lane(x, d, take_min, low)
    return x


def _sort64(x, row, desc):
    for kk, d in _stages(K):
        low = (row & d) == 0
        if kk == K:
            x = _ce_sub_clean(x, d, low, desc)
        else:
            blk = (row & kk) != 0
            m = jnp.logical_xor(low, blk)
            take_max = m if desc else jnp.logical_not(m)
            x = _ce_sub(x, d, take_max, low)
    return x


def _clean64(c, row, desc):
    d = K // 2
    while d >= 1:
        low = (row & d) == 0
        c = _ce_sub_clean(c, d, low, desc)
        d //= 2
    return c


def _merge64(a_desc, b_asc, row, desc):
    return _clean64(jnp.maximum(a_desc, b_asc), row, desc)


def _block_top64(tiles, row, desc):
    if len(tiles) == 1:
        return _sort64(tiles[0], row, desc)
    h = len(tiles) // 2
    a = _block_top64(tiles[:h], row, True)
    b = _block_top64(tiles[h:], row, False)
    return _merge64(a, b, row, desc)


def _tc_body(x_ref, o_ref):
    i = pl.program_id(0)
    x = x_ref[...]
    rowg = jax.lax.broadcasted_iota(jnp.int32, (BLK, 1), 0) + i * BLK
    x = jnp.where(rowg < N, x, NEG)
    lane = jax.lax.broadcasted_iota(jnp.int32, (1, D), 1)
    x = _sort_rows_asc(x, lane)

    row = jax.lax.broadcasted_iota(jnp.int32, (K, 1), 0)
    tiles = [x[t * K:(t + 1) * K, :] for t in range(BLK // K)]
    o_ref[...] = _block_top64(tiles, row, desc=True)


def _run_tc(feat, interpret=False):
    return pl.pallas_call(
        _tc_body,
        grid=(GRID,),
        in_specs=[pl.BlockSpec((BLK, D), lambda i: (i, 0))],
        out_specs=pl.BlockSpec((K, D), lambda i: (i, 0)),
        out_shape=jax.ShapeDtypeStruct((CAND, D), jnp.float32),
        compiler_params=pltpu.CompilerParams(
            dimension_semantics=("parallel",)),
        interpret=interpret,
    )(feat)


def _ce16_desc(z, d):
    i16 = lax.iota(jnp.int32, 16)
    p = jnp.take_along_axis(z, i16 ^ d, axis=0, mode="promise_in_bounds")
    low = (i16 & d) == 0
    return jnp.where(low, jnp.maximum(z, p), jnp.minimum(z, p))


def _sc_merge_desc(best, run):
    rev = [lax.rev(run[3 - t], (0,)) for t in range(4)]
    c = [jnp.maximum(best[t], rev[t]) for t in range(4)]
    y0, y2 = jnp.maximum(c[0], c[2]), jnp.minimum(c[0], c[2])
    y1, y3 = jnp.maximum(c[1], c[3]), jnp.minimum(c[1], c[3])
    z0, z1 = jnp.maximum(y0, y1), jnp.minimum(y0, y1)
    z2, z3 = jnp.maximum(y2, y3), jnp.minimum(y2, y3)
    out = []
    for z in (z0, z1, z2, z3):
        for d in (8, 4, 2, 1):
            z = _ce16_desc(z, d)
        out.append(z)
    return tuple(out)


def _sc_fold(cand_hbm, out_hbm, colbuf, outbuf):
    wid = lax.axis_index("s") * 2 + lax.axis_index("c")
    bc = wid * CH_PER
    pltpu.sync_copy(cand_hbm.at[pl.ds(bc, CH_PER)], colbuf)
    for j in range(CH_PER):
        best = tuple(colbuf[j, 16 * t:16 * (t + 1)] for t in range(4))

        def body(r, b, j=j):
            run = tuple(colbuf[j, pl.ds(r * K + 16 * t, 16)]
                        for t in range(4))
            return _sc_merge_desc(b, run)

        best = lax.fori_loop(1, GRID, body, best)
        for t in range(4):
            outbuf[j, 16 * t:16 * (t + 1)] = best[t]
    pltpu.sync_copy(outbuf, out_hbm.at[pl.ds(bc, CH_PER)])


def _run_sc(cand_t):
    mesh = plsc.VectorSubcoreMesh(core_axis_name="c", subcore_axis_name="s")
    f = pl.kernel(
        _sc_fold,
        out_type=jax.ShapeDtypeStruct((D, K), jnp.float32),
        mesh=mesh,
        scratch_types=[
            pltpu.VMEM((CH_PER, CAND), jnp.float32),
            pltpu.VMEM((CH_PER, K), jnp.float32),
        ],
    )
    return f(cand_t)


@jax.jit
def kernel(feat):
    cand = _run_tc(feat)
    scout = _run_sc(cand.T)
    return scout.T.reshape(K * D)


# speedup vs baseline: 2.3392x; 1.4875x over previous
import jax
import jax.numpy as jnp
from jax import lax
from jax.experimental import pallas as pl
from jax.experimental.pallas import tpu as pltpu
from jax.experimental.pallas import tpu_sc as plsc

N = 100000
D = 128
K = 64
BLK = 2048
GRID = (N + BLK - 1) // BLK
CAND = GRID * K
NSUB = 32
CH_PER = D // NSUB

NEG = float(-jnp.finfo(jnp.float32).max)


def _ce_lane(x, d, take_min, low):
    a = pltpu.roll(x, D - d, 1)
    b = pltpu.roll(x, d, 1)
    xp = jnp.where(low, a, b)
    return jnp.where(take_min, jnp.minimum(x, xp), jnp.maximum(x, xp))


def _ce_sub(x, d, take_max, low):
    a = pltpu.roll(x, x.shape[0] - d, 0)
    b = pltpu.roll(x, d, 0)
    xp = jnp.where(low, a, b)
    return jnp.where(take_max, jnp.maximum(x, xp), jnp.minimum(x, xp))


def _ce_lane_clean(x, d, low):
    a = pltpu.roll(x, D - d, 1)
    b = pltpu.roll(x, d, 1)
    return jnp.where(low, jnp.minimum(x, a), jnp.maximum(x, b))


def _ce_sub_clean(x, d, low, desc):
    a = pltpu.roll(x, x.shape[0] - d, 0)
    b = pltpu.roll(x, d, 0)
    if desc:
        return jnp.where(low, jnp.maximum(x, a), jnp.minimum(x, b))
    return jnp.where(low, jnp.minimum(x, a), jnp.maximum(x, b))


def _stages(n):
    out = []
    kk = 2
    while kk <= n:
        d = kk // 2
        while d >= 1:
            out.append((kk, d))
            d //= 2
        kk *= 2
    return out


def _sort_rows_asc(x, lane):
    for kk, d in _stages(D):
        low = (lane & d) == 0
        idx = jnp.broadcast_to(lane ^ d, x.shape)
        xp = jnp.take_along_axis(x, idx, axis=1, mode="promise_in_bounds")
        if kk == D:
            take_min = low
        else:
            desc = (lane & kk) != 0
            take_min = jnp.logical_xor(low, desc)
        x = jnp.where(take_min, jnp.minimum(x, xp), jnp.maximum(x, xp))
    return x


def _sort64(x, row, desc):
    for kk, d in _stages(K):
        low = (row & d) == 0
        if kk == K:
            x = _ce_sub_clean(x, d, low, desc)
        else:
            blk = (row & kk) != 0
            m = jnp.logical_xor(low, blk)
            take_max = m if desc else jnp.logical_not(m)
            x = _ce_sub(x, d, take_max, low)
    return x


def _clean64(c, row, desc):
    d = K // 2
    while d >= 1:
        low = (row & d) == 0
        c = _ce_sub_clean(c, d, low, desc)
        d //= 2
    return c


def _merge64(a_desc, b_asc, row, desc):
    return _clean64(jnp.maximum(a_desc, b_asc), row, desc)


def _block_top64(tiles, row, desc):
    if len(tiles) == 1:
        return _sort64(tiles[0], row, desc)
    h = len(tiles) // 2
    a = _block_top64(tiles[:h], row, True)
    b = _block_top64(tiles[h:], row, False)
    return _merge64(a, b, row, desc)


def _tc_body(x_ref, o_ref):
    i = pl.program_id(0)
    x = x_ref[...]
    rowg = jax.lax.broadcasted_iota(jnp.int32, (BLK, 1), 0) + i * BLK
    x = jnp.where(rowg < N, x, NEG)
    lane = jax.lax.broadcasted_iota(jnp.int32, (1, D), 1)
    x = _sort_rows_asc(x, lane)

    row = jax.lax.broadcasted_iota(jnp.int32, (K, 1), 0)
    tiles = [x[t * K:(t + 1) * K, :] for t in range(BLK // K)]
    o_ref[...] = _block_top64(tiles, row, desc=True)


def _run_tc(feat, interpret=False):
    return pl.pallas_call(
        _tc_body,
        grid=(GRID,),
        in_specs=[pl.BlockSpec((BLK, D), lambda i: (i, 0))],
        out_specs=pl.BlockSpec((K, D), lambda i: (i, 0)),
        out_shape=jax.ShapeDtypeStruct((CAND, D), jnp.float32),
        compiler_params=pltpu.CompilerParams(
            dimension_semantics=("parallel",)),
        interpret=interpret,
    )(feat)


def _ce16_desc(z, d):
    i16 = lax.iota(jnp.int32, 16)
    p = jnp.take_along_axis(z, i16 ^ d, axis=0, mode="promise_in_bounds")
    low = (i16 & d) == 0
    return jnp.where(low, jnp.maximum(z, p), jnp.minimum(z, p))


def _sc_merge_desc(best, run):
    rev = [lax.rev(run[3 - t], (0,)) for t in range(4)]
    c = [jnp.maximum(best[t], rev[t]) for t in range(4)]
    y0, y2 = jnp.maximum(c[0], c[2]), jnp.minimum(c[0], c[2])
    y1, y3 = jnp.maximum(c[1], c[3]), jnp.minimum(c[1], c[3])
    z0, z1 = jnp.maximum(y0, y1), jnp.minimum(y0, y1)
    z2, z3 = jnp.maximum(y2, y3), jnp.minimum(y2, y3)
    out = []
    for z in (z0, z1, z2, z3):
        for d in (8, 4, 2, 1):
            z = _ce16_desc(z, d)
        out.append(z)
    return tuple(out)


def _sc_fold(cand_hbm, out_hbm, colbuf, outbuf):
    wid = lax.axis_index("s") * 2 + lax.axis_index("c")
    bc = wid * CH_PER
    pltpu.sync_copy(cand_hbm.at[pl.ds(bc, CH_PER)], colbuf)
    for j in range(CH_PER):
        best = tuple(colbuf[j, 16 * t:16 * (t + 1)] for t in range(4))

        def body(r, b, j=j):
            run = tuple(colbuf[j, pl.ds(r * K + 16 * t, 16)]
                        for t in range(4))
            return _sc_merge_desc(b, run)

        best = lax.fori_loop(1, GRID, body, best)
        for t in range(4):
            outbuf[j, 16 * t:16 * (t + 1)] = best[t]
    pltpu.sync_copy(outbuf, out_hbm.at[pl.ds(bc, CH_PER)])


def _run_sc(cand_t):
    mesh = plsc.VectorSubcoreMesh(core_axis_name="c", subcore_axis_name="s")
    f = pl.kernel(
        _sc_fold,
        out_type=jax.ShapeDtypeStruct((D, K), jnp.float32),
        mesh=mesh,
        scratch_types=[
            pltpu.VMEM((CH_PER, CAND), jnp.float32),
            pltpu.VMEM((CH_PER, K), jnp.float32),
        ],
    )
    return f(cand_t)


@jax.jit
def kernel(feat):
    cand = _run_tc(feat)
    scout = _run_sc(cand.T)
    return scout.T.reshape(K * D)


# speedup vs baseline: 2.3395x; 1.0002x over previous
import jax
import jax.numpy as jnp
from jax import lax
from jax.experimental import pallas as pl
from jax.experimental.pallas import tpu as pltpu
from jax.experimental.pallas import tpu_sc as plsc

N = 100000
D = 128
K = 64
BLK = 2048
GRID = (N + BLK - 1) // BLK
CAND = GRID * K
NSUB = 32
CH_PER = D // NSUB

NEG = float(-jnp.finfo(jnp.float32).max)


def _ce_lane(x, d, take_min, low):
    a = pltpu.roll(x, D - d, 1)
    b = pltpu.roll(x, d, 1)
    xp = jnp.where(low, a, b)
    return jnp.where(take_min, jnp.minimum(x, xp), jnp.maximum(x, xp))


def _ce_sub(x, d, take_max, row):
    low = (row & d) == 0
    a = pltpu.roll(x, x.shape[0] - d, 0)
    b = pltpu.roll(x, d, 0)
    xp = jnp.where(low, a, b)
    return jnp.where(take_max, jnp.maximum(x, xp), jnp.minimum(x, xp))


def _ce_lane_clean(x, d, low):
    a = pltpu.roll(x, D - d, 1)
    b = pltpu.roll(x, d, 1)
    return jnp.where(low, jnp.minimum(x, a), jnp.maximum(x, b))


def _ce_sub_clean(x, d, low, desc):
    a = pltpu.roll(x, x.shape[0] - d, 0)
    b = pltpu.roll(x, d, 0)
    if desc:
        return jnp.where(low, jnp.maximum(x, a), jnp.minimum(x, b))
    return jnp.where(low, jnp.minimum(x, a), jnp.maximum(x, b))


def _stages(n):
    out = []
    kk = 2
    while kk <= n:
        d = kk // 2
        while d >= 1:
            out.append((kk, d))
            d //= 2
        kk *= 2
    return out


def _sort_rows_asc(x, lane):
    for kk, d in _stages(D):
        low = (lane & d) == 0
        idx = jnp.broadcast_to(lane ^ d, x.shape)
        xp = jnp.take_along_axis(x, idx, axis=1, mode="promise_in_bounds")
        if kk == D:
            take_min = low
        else:
            desc = (lane & kk) != 0
            take_min = jnp.logical_xor(low, desc)
        x = jnp.where(take_min, jnp.minimum(x, xp), jnp.maximum(x, xp))
    return x


def _sort64(x, row, desc):
    for kk, d in _stages(K):
        low = (row & d) == 0
        if kk == K:
            x = _ce_sub_clean(x, d, low, desc)
        else:
            blk = (row & kk) != 0
            m = jnp.logical_xor(low, blk)
            take_max = m if desc else jnp.logical_not(m)
            x = _ce_sub(x, d, take_max, row)
    return x


def _clean64(c, row, desc):
    d = K // 2
    while d >= 1:
        low = (row & d) == 0
        c = _ce_sub_clean(c, d, low, desc)
        d //= 2
    return c


def _merge64(a_desc, b_asc, row, desc):
    return _clean64(jnp.maximum(a_desc, b_asc), row, desc)


def _block_top64(tiles, row, desc):
    if len(tiles) == 1:
        return _sort64(tiles[0], row, desc)
    h = len(tiles) // 2
    a = _block_top64(tiles[:h], row, True)
    b = _block_top64(tiles[h:], row, False)
    return _merge64(a, b, row, desc)


def _tc_body(x_ref, o_ref):
    i = pl.program_id(0)
    x = x_ref[...]
    rowg = jax.lax.broadcasted_iota(jnp.int32, (BLK, 1), 0) + i * BLK
    x = jnp.where(rowg < N, x, NEG)
    lane = jax.lax.broadcasted_iota(jnp.int32, (1, D), 1)
    x = _sort_rows_asc(x, lane)

    row = jax.lax.broadcasted_iota(jnp.int32, (K, 1), 0)
    tiles = [x[t * K:(t + 1) * K, :] for t in range(BLK // K)]
    o_ref[...] = _block_top64(tiles, row, desc=True)


def _run_tc(feat, interpret=False):
    return pl.pallas_call(
        _tc_body,
        grid=(GRID,),
        in_specs=[pl.BlockSpec((BLK, D), lambda i: (i, 0))],
        out_specs=pl.BlockSpec((K, D), lambda i: (i, 0)),
        out_shape=jax.ShapeDtypeStruct((CAND, D), jnp.float32),
        compiler_params=pltpu.CompilerParams(
            dimension_semantics=("parallel",)),
        interpret=interpret,
    )(feat)


def _ce16_desc(z, d):
    i16 = lax.iota(jnp.int32, 16)
    p = jnp.take_along_axis(z, i16 ^ d, axis=0, mode="promise_in_bounds")
    low = (i16 & d) == 0
    return jnp.where(low, jnp.maximum(z, p), jnp.minimum(z, p))


def _sc_merge_desc(best, run):
    rev = [lax.rev(run[3 - t], (0,)) for t in range(4)]
    c = [jnp.maximum(best[t], rev[t]) for t in range(4)]
    y0, y2 = jnp.maximum(c[0], c[2]), jnp.minimum(c[0], c[2])
    y1, y3 = jnp.maximum(c[1], c[3]), jnp.minimum(c[1], c[3])
    z0, z1 = jnp.maximum(y0, y1), jnp.minimum(y0, y1)
    z2, z3 = jnp.maximum(y2, y3), jnp.minimum(y2, y3)
    out = []
    for z in (z0, z1, z2, z3):
        for d in (8, 4, 2, 1):
            z = _ce16_desc(z, d)
        out.append(z)
    return tuple(out)


def _sc_fold(cand_hbm, out_hbm, colbuf, outbuf):
    wid = lax.axis_index("s") * 2 + lax.axis_index("c")
    bc = wid * CH_PER
    pltpu.sync_copy(cand_hbm.at[pl.ds(bc, CH_PER)], colbuf)
    for j in range(CH_PER):
        best = tuple(colbuf[j, 16 * t:16 * (t + 1)] for t in range(4))

        def body(r, b, j=j):
            run = tuple(colbuf[j, pl.ds(r * K + 16 * t, 16)]
                        for t in range(4))
            return _sc_merge_desc(b, run)

        best = lax.fori_loop(1, GRID, body, best)
        for t in range(4):
            outbuf[j, 16 * t:16 * (t + 1)] = best[t]
    pltpu.sync_copy(outbuf, out_hbm.at[pl.ds(bc, CH_PER)])


def _run_sc(cand_t):
    mesh = plsc.VectorSubcoreMesh(core_axis_name="c", subcore_axis_name="s")
    f = pl.kernel(
        _sc_fold,
        out_type=jax.ShapeDtypeStruct((D, K), jnp.float32),
        mesh=mesh,
        scratch_types=[
            pltpu.VMEM((CH_PER, CAND), jnp.float32),
            pltpu.VMEM((CH_PER, K), jnp.float32),
        ],
    )
    return f(cand_t)


@jax.jit
def kernel(feat):
    cand = _run_tc(feat)
    scout = _run_sc(cand.T)
    return scout.T.reshape(K * D)
